# baseline (speedup 1.0000x reference)
import functools

import jax
import jax.numpy as jnp
from jax import lax
from jax.experimental import pallas as pl
from jax.experimental.pallas import tpu as pltpu
from jax.experimental.pallas import tpu_sc as plsc

EMBED_DIM = 64
IDX_CHUNK = 128
CHUNKS_PER_STEP = 4


def _make_gather_kernel(batch):
    n_chunks = batch // IDX_CHUNK
    n_steps = n_chunks // CHUNKS_PER_STEP
    mesh = plsc.VectorSubcoreMesh(core_axis_name="c", subcore_axis_name="s")

    @functools.partial(
        pl.kernel,
        out_type=jax.ShapeDtypeStruct((2 * EMBED_DIM, batch), jnp.float32),
        mesh=mesh,
        compiler_params=pltpu.CompilerParams(use_tc_tiling_on_sc=False),
        scratch_types=[
            pltpu.VMEM((n_chunks, IDX_CHUNK), jnp.int32),
            pltpu.VMEM((n_chunks, IDX_CHUNK), jnp.int32),
            pltpu.VMEM((batch,), jnp.float32),
            pltpu.VMEM((batch,), jnp.float32),
            pltpu.VMEM((batch,), jnp.float32),
            pltpu.VMEM((batch,), jnp.float32),
            pltpu.SemaphoreType.DMA,
        ],
    )
    def gather_kernel(uidx_hbm, aidx_hbm, utabT_hbm, atabT_hbm, outT_hbm,
                      uidx_v, aidx_v, u0_v, u1_v, a0_v, a1_v, sem):
        wid = lax.axis_index("s") * 2 + lax.axis_index("c")
        d0 = wid * 2
        pltpu.sync_copy(uidx_hbm, uidx_v)
        pltpu.sync_copy(aidx_hbm, aidx_v)

        def step(it, carry):
            copies = []
            for cc in range(CHUNKS_PER_STEP):
                c = it * CHUNKS_PER_STEP + cc
                dst = pl.ds(c * IDX_CHUNK, IDX_CHUNK)
                copies.append(pltpu.async_copy(
                    utabT_hbm.at[d0].at[uidx_v.at[c]], u0_v.at[dst], sem))
                copies.append(pltpu.async_copy(
                    utabT_hbm.at[d0 + 1].at[uidx_v.at[c]], u1_v.at[dst], sem))
                copies.append(pltpu.async_copy(
                    atabT_hbm.at[d0].at[aidx_v.at[c]], a0_v.at[dst], sem))
                copies.append(pltpu.async_copy(
                    atabT_hbm.at[d0 + 1].at[aidx_v.at[c]], a1_v.at[dst], sem))
            for c_ in copies:
                c_.wait()
            return carry

        lax.fori_loop(0, n_steps, step, 0)
        pltpu.sync_copy(u0_v, outT_hbm.at[d0])
        pltpu.sync_copy(u1_v, outT_hbm.at[d0 + 1])
        pltpu.sync_copy(a0_v, outT_hbm.at[EMBED_DIM + d0])
        pltpu.sync_copy(a1_v, outT_hbm.at[EMBED_DIM + d0 + 1])

    return gather_kernel


def _mlp_body(x_ref, w1_ref, b1_ref, w2_ref, b2_ref, w3_ref, out_ref):
    xT = x_ref[...]
    h1 = lax.dot_general(w1_ref[...], xT, (((0,), (0,)), ((), ())),
                         preferred_element_type=jnp.float32)
    h1 = jnp.maximum(h1 + b1_ref[...], 0.0)
    h2 = lax.dot_general(w2_ref[...], h1, (((0,), (0,)), ((), ())),
                         preferred_element_type=jnp.float32)
    h2 = jnp.maximum(h2 + b2_ref[...], 0.0)
    out_ref[...] = jnp.sum(h2 * w3_ref[...], axis=0)


def _mlp(xT, W1, b1, W2, b2, W3, block_b):
    batch = xT.shape[1]
    grid = (batch // block_b,)
    full = lambda i: (0, 0)
    out = pl.pallas_call(
        _mlp_body,
        grid=grid,
        in_specs=[
            pl.BlockSpec((2 * EMBED_DIM, block_b), lambda i: (0, i)),
            pl.BlockSpec((128, 128), full),
            pl.BlockSpec((128, 1), full),
            pl.BlockSpec((128, EMBED_DIM), full),
            pl.BlockSpec((EMBED_DIM, 1), full),
            pl.BlockSpec((EMBED_DIM, 1), full),
        ],
        out_specs=pl.BlockSpec((block_b,), lambda i: (i,)),
        out_shape=jax.ShapeDtypeStruct((batch,), jnp.float32),
    )(xT, W1, b1.reshape(128, 1), W2, b2.reshape(EMBED_DIM, 1), W3)
    return out


def kernel(user_id, anime_id, user_table, anime_table, W1, b1, W2, b2, W3, b3):
    batch = user_id.shape[0]
    gk = _make_gather_kernel(batch)
    xT = gk(user_id.reshape(-1, IDX_CHUNK), anime_id.reshape(-1, IDX_CHUNK),
            user_table.T, anime_table.T)
    out = _mlp(xT, W1, b1, W2, b2, W3, block_b=2048)
    return out[:, None] + b3


# speedup vs baseline: 3.6186x; 3.6186x over previous
import functools

import jax
import jax.numpy as jnp
from jax import lax
from jax.experimental import pallas as pl
from jax.experimental.pallas import tpu as pltpu
from jax.experimental.pallas import tpu_sc as plsc

EMBED_DIM = 64
PAIR_DIM = 2 * EMBED_DIM
IDX_CHUNK = 128
NW = 32
NSLOT = 4


def _cdiv(a, b):
    return (a + b - 1) // b


def _relayout_table(tabT_hbm, tab2_hbm, n_panels, wid,
                    wbuf, obuf, lsems, osems, rowvecs, colvecs):
    ppw = _cdiv(_cdiv(n_panels, NW), NSLOT) * NSLOT
    p0 = wid * ppw
    dump_panel = n_panels

    def fire_load(b, p):
        pc = jnp.minimum(p, n_panels - 1)
        pltpu.async_copy(
            tabT_hbm.at[:, pl.ds(pl.multiple_of(pc * IDX_CHUNK, IDX_CHUNK),
                                 IDX_CHUNK)],
            wbuf.at[b], lsems[b])

    def fire_out(b, p):
        row = jnp.where(p < n_panels, p, dump_panel) * 64
        pltpu.async_copy(obuf.at[b], tab2_hbm.at[pl.ds(row, 64)], osems[b])

    for b in range(NSLOT):
        fire_load(b, p0 + b)
        fire_out(b, jnp.int32(dump_panel))

    def step(it, carry):
        for b in range(NSLOT):
            p = p0 + it * NSLOT + b
            pltpu.make_async_copy(
                tabT_hbm.at[:, pl.ds(0, IDX_CHUNK)],
                wbuf.at[b], lsems[b]).wait()
            pltpu.make_async_copy(
                obuf.at[b], tab2_hbm.at[pl.ds(0, 64)], osems[b]).wait()

            def drow(d, c2):
                for g in range(8):
                    val = wbuf.at[b].at[d][pl.ds(g * 16, 16)]
                    plsc.store_scatter(
                        obuf.at[b], [rowvecs[g], colvecs[g] + d], val)
                return c2

            lax.fori_loop(0, EMBED_DIM, drow, 0)
            fire_out(b, p)
            fire_load(b, p + NSLOT)
        return carry

    lax.fori_loop(0, ppw // NSLOT, step, 0)
    for b in range(NSLOT):
        pltpu.make_async_copy(
            tabT_hbm.at[:, pl.ds(0, IDX_CHUNK)],
            wbuf.at[b], lsems[b]).wait()
        pltpu.make_async_copy(
            obuf.at[b], tab2_hbm.at[pl.ds(0, 64)], osems[b]).wait()


def _make_relayout_kernel(n_user, n_anime):
    u_panels = _cdiv(n_user, IDX_CHUNK)
    a_panels = _cdiv(n_anime, IDX_CHUNK)
    u_rows = u_panels * 64 + 64
    a_rows = a_panels * 64 + 64
    mesh = plsc.VectorSubcoreMesh(core_axis_name="c", subcore_axis_name="s")

    @functools.partial(
        pl.kernel,
        out_type=(
            jax.ShapeDtypeStruct((u_rows, PAIR_DIM), jnp.float32),
            jax.ShapeDtypeStruct((a_rows, PAIR_DIM), jnp.float32),
        ),
        compiler_params=pltpu.CompilerParams(needs_layout_passes=False),
        mesh=mesh,
        scratch_types=[
            pltpu.VMEM((NSLOT, EMBED_DIM, IDX_CHUNK), jnp.float32),
            pltpu.VMEM((NSLOT, 64, PAIR_DIM), jnp.float32),
        ] + [pltpu.SemaphoreType.DMA] * (2 * NSLOT),
    )
    def relayout_kernel(utabT_hbm, atabT_hbm, utab2_hbm, atab2_hbm,
                        wbuf, obuf, *sems):
        wid = lax.axis_index("s") * 2 + lax.axis_index("c")
        lsems, osems = sems[:NSLOT], sems[NSLOT:]
        rowvecs = []
        colvecs = []
        for g in range(8):
            lanes = g * 16 + lax.iota(jnp.int32, 16)
            rowvecs.append(lanes // 2)
            colvecs.append((lanes % 2) * EMBED_DIM)
        _relayout_table(utabT_hbm, utab2_hbm, u_panels, wid,
                        wbuf, obuf, lsems, osems, rowvecs, colvecs)
        _relayout_table(atabT_hbm, atab2_hbm, a_panels, wid,
                        wbuf, obuf, lsems, osems, rowvecs, colvecs)

    return relayout_kernel


def _make_gather_kernel(batch):
    chunks_per_worker = batch // (NW * IDX_CHUNK)
    rows_per_worker = chunks_per_worker * IDX_CHUNK
    mesh = plsc.VectorSubcoreMesh(core_axis_name="c", subcore_axis_name="s")

    @functools.partial(
        pl.kernel,
        out_type=(
            jax.ShapeDtypeStruct((batch, PAIR_DIM), jnp.float32),
            jax.ShapeDtypeStruct((batch, PAIR_DIM), jnp.float32),
        ),
        mesh=mesh,
        scratch_types=[
            pltpu.VMEM((batch // (NW * IDX_CHUNK), IDX_CHUNK), jnp.int32),
            pltpu.VMEM((batch // (NW * IDX_CHUNK), IDX_CHUNK), jnp.int32),
            pltpu.VMEM((batch // NW, PAIR_DIM), jnp.float32),
            pltpu.SemaphoreType.DMA,
        ],
    )
    def gather_kernel(uidx_hbm, aidx_hbm, utab2_hbm, atab2_hbm,
                      uout_hbm, aout_hbm,
                      uidx_v, aidx_v, rows_v, sem):
        wid = lax.axis_index("s") * 2 + lax.axis_index("c")
        crow = wid * chunks_per_worker
        base = wid * rows_per_worker
        pltpu.sync_copy(uidx_hbm.at[pl.ds(crow, chunks_per_worker)], uidx_v)
        pltpu.sync_copy(aidx_hbm.at[pl.ds(crow, chunks_per_worker)], aidx_v)
        copies = []
        for j in range(chunks_per_worker):
            copies.append(pltpu.async_copy(
                utab2_hbm.at[uidx_v.at[j]],
                rows_v.at[pl.ds(j * IDX_CHUNK, IDX_CHUNK)], sem))
        for c in copies:
            c.wait()
        pltpu.sync_copy(rows_v, uout_hbm.at[pl.ds(base, rows_per_worker)])
        copies = []
        for j in range(chunks_per_worker):
            copies.append(pltpu.async_copy(
                atab2_hbm.at[aidx_v.at[j]],
                rows_v.at[pl.ds(j * IDX_CHUNK, IDX_CHUNK)], sem))
        for c in copies:
            c.wait()
        pltpu.sync_copy(rows_v, aout_hbm.at[pl.ds(base, rows_per_worker)])

    return gather_kernel


def _mlp_body(upair_ref, apair_ref, uid_ref, aid_ref,
              w1u_ref, w1a_ref, b1_ref, w2_ref, b2_ref, w3_ref, out_ref):
    up = upair_ref[...]
    ap = apair_ref[...]
    usel = (uid_ref[...] & 1) == 1
    asel = (aid_ref[...] & 1) == 1
    u = jnp.where(usel, up[:, EMBED_DIM:], up[:, :EMBED_DIM])
    a = jnp.where(asel, ap[:, EMBED_DIM:], ap[:, :EMBED_DIM])
    h1 = jnp.dot(u, w1u_ref[...], preferred_element_type=jnp.float32)
    h1 = h1 + jnp.dot(a, w1a_ref[...], preferred_element_type=jnp.float32)
    h1 = jnp.maximum(h1 + b1_ref[...], 0.0)
    h2 = jnp.dot(h1, w2_ref[...], preferred_element_type=jnp.float32)
    h2 = jnp.maximum(h2 + b2_ref[...], 0.0)
    out_ref[...] = jnp.sum(h2 * w3_ref[...], axis=1)


def _mlp(upairs, apairs, user_id, anime_id, W1, b1, W2, b2, W3, block_b):
    batch = upairs.shape[0]
    grid = (batch // block_b,)
    full = lambda i: (0, 0)
    out = pl.pallas_call(
        _mlp_body,
        grid=grid,
        in_specs=[
            pl.BlockSpec((block_b, PAIR_DIM), lambda i: (i, 0)),
            pl.BlockSpec((block_b, PAIR_DIM), lambda i: (i, 0)),
            pl.BlockSpec((block_b, 1), lambda i: (i, 0)),
            pl.BlockSpec((block_b, 1), lambda i: (i, 0)),
            pl.BlockSpec((EMBED_DIM, 128), full),
            pl.BlockSpec((EMBED_DIM, 128), full),
            pl.BlockSpec((1, 128), full),
            pl.BlockSpec((128, EMBED_DIM), full),
            pl.BlockSpec((1, EMBED_DIM), full),
            pl.BlockSpec((1, EMBED_DIM), full),
        ],
        out_specs=pl.BlockSpec((block_b,), lambda i: (i,)),
        out_shape=jax.ShapeDtypeStruct((batch,), jnp.float32),
    )(upairs, apairs, user_id[:, None], anime_id[:, None],
      W1[:EMBED_DIM], W1[EMBED_DIM:],
      b1.reshape(1, 128), W2, b2.reshape(1, EMBED_DIM),
      W3.reshape(1, EMBED_DIM))
    return out


def kernel(user_id, anime_id, user_table, anime_table, W1, b1, W2, b2, W3, b3):
    batch = user_id.shape[0]
    n_user, n_anime = user_table.shape[0], anime_table.shape[0]
    rk = _make_relayout_kernel(n_user, n_anime)
    utab2, atab2 = rk(user_table.T, anime_table.T)
    gk = _make_gather_kernel(batch)
    upairs, apairs = gk((user_id // 2).reshape(-1, IDX_CHUNK),
                        (anime_id // 2).reshape(-1, IDX_CHUNK),
                        utab2, atab2)
    out = _mlp(upairs, apairs, user_id, anime_id,
               W1, b1, W2, b2, W3, block_b=2048)
    return out[:, None] + b3
